# Initial kernel scaffold; baseline (speedup 1.0000x reference)
import jax
import jax.numpy as jnp
from jax.experimental import pallas as pl


def kernel(logits, targets):
    raise NotImplementedError("write your pallas kernel here")


# speedup vs baseline: 62.0490x; 62.0490x over previous
import functools

import jax
import jax.numpy as jnp
from jax import lax
from jax.experimental import pallas as pl
from jax.experimental.pallas import tpu as pltpu
from jax.experimental.pallas import tpu_sc as plsc

B, C, H, W = 4, 19, 512, 512
M = 2048
NBINS = 2 * C * M
NKEYS = B * C * H * W
NW = 32
PER_TILE = NKEYS // NW
CHUNK = 4096
NCHUNKS = PER_TILE // CHUNK


def _keys_body(logits_ref, targets_ref, keys_ref):
    x = logits_ref[...]
    m = jnp.max(x, axis=1, keepdims=True)
    ex = jnp.exp(x - m)
    p = ex / jnp.sum(ex, axis=1, keepdims=True)
    lab = targets_ref[...]
    cidx = lax.broadcasted_iota(jnp.int32, p.shape, 1)
    fg = lab[:, None, :, :] == cidx
    err = jnp.abs(fg.astype(jnp.float32) - p)
    bins = jnp.minimum((err * M).astype(jnp.int32), M - 1)
    keys_ref[...] = jnp.where(fg, C * M, 0) + cidx * M + bins


def _make_keys(logits, targets):
    hb = 64
    grid = (B, H // hb)
    return pl.pallas_call(
        _keys_body,
        grid=grid,
        in_specs=[
            pl.BlockSpec((1, C, hb, W), lambda b, h: (b, 0, h, 0)),
            pl.BlockSpec((1, hb, W), lambda b, h: (b, h, 0)),
        ],
        out_specs=pl.BlockSpec((1, C, hb, W), lambda b, h: (b, 0, h, 0)),
        out_shape=jax.ShapeDtypeStruct((B, C, H, W), jnp.int32),
    )(logits, targets)


def _hist_body(keys_hbm, out_hbm, kbuf, hist_v):
    wid = lax.axis_index("s") * 2 + lax.axis_index("c")
    base = wid * PER_TILE

    def _zero(i, _):
        hist_v[pl.ds(i * 16, 16)] = jnp.zeros((16,), jnp.int32)
        return 0

    lax.fori_loop(0, NBINS // 16, _zero, 0)

    ones = jnp.ones((16,), jnp.int32)

    def _chunk(g, _):
        pltpu.sync_copy(keys_hbm.at[pl.ds(base + g * CHUNK, CHUNK)], kbuf)

        def _vec(i, _):
            k = kbuf[pl.ds(i * 16, 16)]
            plsc.addupdate_scatter(hist_v, [k], ones)
            return 0

        lax.fori_loop(0, CHUNK // 16, _vec, 0)
        return 0

    lax.fori_loop(0, NCHUNKS, _chunk, 0)
    pltpu.sync_copy(hist_v, out_hbm.at[wid])


def _histogram(keys_flat):
    mesh = plsc.VectorSubcoreMesh(core_axis_name="c", subcore_axis_name="s")
    fn = functools.partial(
        pl.kernel,
        mesh=mesh,
        out_type=jax.ShapeDtypeStruct((NW, NBINS), jnp.int32),
        scratch_types=[
            pltpu.VMEM((CHUNK,), jnp.int32),
            pltpu.VMEM((NBINS,), jnp.int32),
        ],
        compiler_params=pltpu.CompilerParams(needs_layout_passes=False),
    )(_hist_body)
    return fn(keys_flat)


def _final_body(hist_ref, out_ref):
    h = jnp.sum(hist_ref[...], axis=0).astype(jnp.float32)
    hfg = h[1]
    htot = h[0] + hfg
    x = jnp.concatenate([htot, hfg], axis=0)
    rows = lax.broadcasted_iota(jnp.int32, (M, M), 0)
    cols = lax.broadcasted_iota(jnp.int32, (M, M), 1)
    tri = (rows >= cols).astype(jnp.float32)
    cum = jnp.dot(x, tri, preferred_element_type=jnp.float32)
    cumN = cum[:C]
    cumP = cum[C:]
    gts = cumP[:, 0:1]
    union = jnp.maximum(gts + cumN - cumP, 1.0)
    jac = jnp.where(cumN > 0, 1.0 - (gts - cumP) / union, 0.0)
    loss_c = (jnp.sum(jac, axis=1) - 0.5 * jac[:, 0]) * (1.0 / M)
    out_ref[...] = jnp.mean(loss_c)[None, None]


def _finalize(hist):
    return pl.pallas_call(
        _final_body,
        out_shape=jax.ShapeDtypeStruct((1, 1), jnp.float32),
    )(hist)


def kernel(logits, targets):
    keys = _make_keys(logits, targets)
    hist = _histogram(keys.reshape(-1))
    out = _finalize(hist.reshape(NW, 2, C, M))
    return out[0, 0]


# speedup vs baseline: 81.5882x; 1.3149x over previous
import functools

import jax
import jax.numpy as jnp
from jax import lax
from jax.experimental import pallas as pl
from jax.experimental.pallas import tpu as pltpu
from jax.experimental.pallas import tpu_sc as plsc

B, C, H, W = 4, 19, 512, 512
M = 2048
NBINS = 2 * C * M
NKEYS = B * C * H * W
NW = 32
PER_TILE = NKEYS // NW
CHUNK = 4096
NCHUNKS = PER_TILE // CHUNK


def _keys_body(logits_ref, targets_ref, keys_ref):
    x = logits_ref[...]
    m = jnp.max(x, axis=1, keepdims=True)
    ex = jnp.exp(x - m)
    p = ex / jnp.sum(ex, axis=1, keepdims=True)
    lab = targets_ref[...]
    cidx = lax.broadcasted_iota(jnp.int32, p.shape, 1)
    fg = lab[:, None, :, :] == cidx
    err = jnp.abs(fg.astype(jnp.float32) - p)
    bins = jnp.minimum((err * M).astype(jnp.int32), M - 1)
    keys_ref[...] = jnp.where(fg, C * M, 0) + cidx * M + bins


def _make_keys(logits, targets):
    hb = 64
    grid = (B, H // hb)
    return pl.pallas_call(
        _keys_body,
        grid=grid,
        in_specs=[
            pl.BlockSpec((1, C, hb, W), lambda b, h: (b, 0, h, 0)),
            pl.BlockSpec((1, hb, W), lambda b, h: (b, h, 0)),
        ],
        out_specs=pl.BlockSpec((1, C, hb, W), lambda b, h: (b, 0, h, 0)),
        out_shape=jax.ShapeDtypeStruct((B, C, H, W), jnp.int32),
    )(logits, targets)


_UNROLL = 8


def _hist_body(keys_hbm, out_hbm, kbuf0, kbuf1, hist_v, sem0, sem1):
    wid = lax.axis_index("s") * 2 + lax.axis_index("c")
    base = wid * PER_TILE

    def _zero(i, _):
        hist_v[pl.ds(i * 16, 16)] = jnp.zeros((16,), jnp.int32)
        return 0

    lax.fori_loop(0, NBINS // 16, _zero, 0)

    ones = jnp.ones((16,), jnp.int32)
    bufs = (kbuf0, kbuf1)
    sems = (sem0, sem1)

    def _start(g, slot):
        pltpu.async_copy(
            keys_hbm.at[pl.ds(base + g * CHUNK, CHUNK)], bufs[slot], sems[slot]
        )

    def _drain(slot):
        pltpu.make_async_copy(
            keys_hbm.at[pl.ds(base, CHUNK)], bufs[slot], sems[slot]
        ).wait()

    def _scan(slot):
        buf = bufs[slot]

        def _vec(i, _):
            for j in range(_UNROLL):
                k = buf[pl.ds((i * _UNROLL + j) * 16, 16)]
                plsc.addupdate_scatter(hist_v, [k], ones)
            return 0

        lax.fori_loop(0, CHUNK // (16 * _UNROLL), _vec, 0)

    _start(0, 0)

    def _pair(p, _):
        g = p * 2
        _drain(0)
        _start(g + 1, 1)
        _scan(0)
        _drain(1)

        @pl.when(g + 2 < NCHUNKS)
        def _():
            _start(g + 2, 0)

        _scan(1)
        return 0

    lax.fori_loop(0, NCHUNKS // 2, _pair, 0)
    pltpu.sync_copy(hist_v, out_hbm.at[wid])


def _histogram(keys_flat):
    mesh = plsc.VectorSubcoreMesh(core_axis_name="c", subcore_axis_name="s")
    fn = functools.partial(
        pl.kernel,
        mesh=mesh,
        out_type=jax.ShapeDtypeStruct((NW, NBINS), jnp.int32),
        scratch_types=[
            pltpu.VMEM((CHUNK,), jnp.int32),
            pltpu.VMEM((CHUNK,), jnp.int32),
            pltpu.VMEM((NBINS,), jnp.int32),
            pltpu.SemaphoreType.DMA,
            pltpu.SemaphoreType.DMA,
        ],
        compiler_params=pltpu.CompilerParams(needs_layout_passes=False),
    )(_hist_body)
    return fn(keys_flat)


def _final_body(hist_ref, out_ref):
    h = jnp.sum(hist_ref[...], axis=0).astype(jnp.float32)
    hfg = h[1]
    htot = h[0] + hfg
    x = jnp.concatenate([htot, hfg], axis=0)
    rows = lax.broadcasted_iota(jnp.int32, (M, M), 0)
    cols = lax.broadcasted_iota(jnp.int32, (M, M), 1)
    tri = (rows >= cols).astype(jnp.float32)
    cum = jnp.dot(x, tri, preferred_element_type=jnp.float32)
    cumN = cum[:C]
    cumP = cum[C:]
    gts = cumP[:, 0:1]
    union = jnp.maximum(gts + cumN - cumP, 1.0)
    jac = jnp.where(cumN > 0, 1.0 - (gts - cumP) / union, 0.0)
    loss_c = (jnp.sum(jac, axis=1) - 0.5 * jac[:, 0]) * (1.0 / M)
    out_ref[...] = jnp.mean(loss_c)[None, None]


def _finalize(hist):
    return pl.pallas_call(
        _final_body,
        out_shape=jax.ShapeDtypeStruct((1, 1), jnp.float32),
    )(hist)


def kernel(logits, targets):
    keys = _make_keys(logits, targets)
    hist = _histogram(keys.reshape(-1))
    out = _finalize(hist.reshape(NW, 2, C, M))
    return out[0, 0]
